# baseline (speedup 1.0000x reference)
import functools

import jax
import jax.numpy as jnp
from jax import lax
from jax.experimental import pallas as pl
from jax.experimental.pallas import tpu as pltpu
from jax.experimental.pallas import tpu_sc as plsc

CAM_LAMBDA = 3.0

B_T = 128
NUM_PATCHES = 196
EMBED_DIM = 768
SEQ = NUM_PATCHES + 1

_SC_WORKERS = 16
_ROWS_PER_W = B_T // _SC_WORKERS


def _sc_cam_gather(table, idx):
    mesh = plsc.VectorSubcoreMesh(
        core_axis_name="c", subcore_axis_name="s", num_cores=1
    )

    @functools.partial(
        pl.kernel,
        mesh=mesh,
        out_type=jax.ShapeDtypeStruct((B_T, EMBED_DIM), jnp.float32),
        scratch_types=[
            pltpu.VMEM((B_T,), jnp.int32),
            pltpu.VMEM((B_T, EMBED_DIM), jnp.float32),
            pltpu.SemaphoreType.DMA,
        ],
    )
    def gather_kernel(table_hbm, idx_hbm, out_hbm, idx_v, rows_v, sem):
        @pl.when(lax.axis_index("s") == 0)
        def _():
            pltpu.sync_copy(idx_hbm, idx_v)
            pltpu.async_copy(table_hbm.at[idx_v], rows_v, sem).wait()
            pltpu.sync_copy(rows_v, out_hbm)

    return gather_kernel(table, idx)


_BB = 8


def _tc_fuse(img_t, heat_t, cam3, cls2d, pos3):
    def body(img_ref, heat_ref, cam_ref, cls_ref, pos_ref, out_ref):
        camv = (cam_ref[...] * CAM_LAMBDA)[None]
        out_ref[0:1] = cls_ref[...][None] + pos_ref[0:1] + camv
        out_ref[1:] = img_ref[...] + heat_ref[...] + pos_ref[1:] + camv

    return pl.pallas_call(
        body,
        grid=(B_T // _BB,),
        in_specs=[
            pl.BlockSpec((NUM_PATCHES, _BB, EMBED_DIM), lambda i: (0, i, 0)),
            pl.BlockSpec((NUM_PATCHES, _BB, EMBED_DIM), lambda i: (0, i, 0)),
            pl.BlockSpec((_BB, EMBED_DIM), lambda i: (i, 0)),
            pl.BlockSpec((1, EMBED_DIM), lambda i: (0, 0)),
            pl.BlockSpec((SEQ, 1, EMBED_DIM), lambda i: (0, 0, 0)),
        ],
        out_specs=pl.BlockSpec((SEQ, _BB, EMBED_DIM), lambda i: (0, i, 0)),
        out_shape=jax.ShapeDtypeStruct((SEQ, B_T, EMBED_DIM), jnp.float32),
        compiler_params=pltpu.CompilerParams(
            dimension_semantics=("parallel",),
        ),
    )(img_t, heat_t, cam3, cls2d, pos3)


def kernel(img_tokens, heatmap_tokens, cam_labels, cls_token, pos_embed, cam):
    labels = cam_labels.astype(jnp.int32)
    cam3 = _sc_cam_gather(cam.reshape(cam.shape[0], cam.shape[-1]), labels)
    img_t = jnp.transpose(img_tokens, (1, 0, 2))
    heat_t = jnp.transpose(heatmap_tokens, (1, 0, 2))
    pos3 = pos_embed.reshape(SEQ, 1, EMBED_DIM)
    cls2d = cls_token.reshape(1, EMBED_DIM)
    out_t = _tc_fuse(img_t, heat_t, cam3, cls2d, pos3)
    return jnp.transpose(out_t, (1, 0, 2))


# speedup vs baseline: 1.0687x; 1.0687x over previous
import functools

import jax
import jax.numpy as jnp
from jax import lax
from jax.experimental import pallas as pl
from jax.experimental.pallas import tpu as pltpu
from jax.experimental.pallas import tpu_sc as plsc

CAM_LAMBDA = 3.0

B_T = 128
NUM_PATCHES = 196
EMBED_DIM = 768
SEQ = NUM_PATCHES + 1

_SC_WORKERS = 16
_ROWS_PER_W = B_T // _SC_WORKERS


def _sc_cam_gather(table, idx):
    mesh = plsc.VectorSubcoreMesh(
        core_axis_name="c", subcore_axis_name="s", num_cores=1
    )

    @functools.partial(
        pl.kernel,
        mesh=mesh,
        out_type=jax.ShapeDtypeStruct((B_T, EMBED_DIM), jnp.float32),
        scratch_types=[
            pltpu.VMEM((_ROWS_PER_W,), jnp.int32),
            pltpu.VMEM((_ROWS_PER_W, EMBED_DIM), jnp.float32),
            pltpu.SemaphoreType.DMA,
        ],
    )
    def gather_kernel(table_hbm, idx_hbm, out_hbm, idx_v, rows_v, sem):
        base = lax.axis_index("s") * _ROWS_PER_W
        pltpu.sync_copy(idx_hbm.at[pl.ds(base, _ROWS_PER_W)], idx_v)
        pltpu.async_copy(table_hbm.at[idx_v], rows_v, sem).wait()
        pltpu.sync_copy(rows_v, out_hbm.at[pl.ds(base, _ROWS_PER_W)])

    return gather_kernel(table, idx)


_BB = 8


def _tc_fuse(img_t, heat_t, cam3, cls2d, pos3):
    def body(img_ref, heat_ref, cam_ref, cls_ref, pos_ref, out_ref):
        camv = (cam_ref[...] * CAM_LAMBDA)[None]
        out_ref[0:1] = cls_ref[...][None] + pos_ref[0:1] + camv
        out_ref[1:] = img_ref[...] + heat_ref[...] + pos_ref[1:] + camv

    return pl.pallas_call(
        body,
        grid=(B_T // _BB,),
        in_specs=[
            pl.BlockSpec((NUM_PATCHES, _BB, EMBED_DIM), lambda i: (0, i, 0)),
            pl.BlockSpec((NUM_PATCHES, _BB, EMBED_DIM), lambda i: (0, i, 0)),
            pl.BlockSpec((_BB, EMBED_DIM), lambda i: (i, 0)),
            pl.BlockSpec((1, EMBED_DIM), lambda i: (0, 0)),
            pl.BlockSpec((SEQ, 1, EMBED_DIM), lambda i: (0, 0, 0)),
        ],
        out_specs=pl.BlockSpec((SEQ, _BB, EMBED_DIM), lambda i: (0, i, 0)),
        out_shape=jax.ShapeDtypeStruct((SEQ, B_T, EMBED_DIM), jnp.float32),
        compiler_params=pltpu.CompilerParams(
            dimension_semantics=("parallel",),
        ),
    )(img_t, heat_t, cam3, cls2d, pos3)


def kernel(img_tokens, heatmap_tokens, cam_labels, cls_token, pos_embed, cam):
    labels = cam_labels.astype(jnp.int32)
    cam3 = _sc_cam_gather(cam.reshape(cam.shape[0], cam.shape[-1]), labels)
    img_t = jnp.transpose(img_tokens, (1, 0, 2))
    heat_t = jnp.transpose(heatmap_tokens, (1, 0, 2))
    pos3 = pos_embed.reshape(SEQ, 1, EMBED_DIM)
    cls2d = cls_token.reshape(1, EMBED_DIM)
    out_t = _tc_fuse(img_t, heat_t, cam3, cls2d, pos3)
    return jnp.transpose(out_t, (1, 0, 2))


# speedup vs baseline: 1.0694x; 1.0006x over previous
import functools

import jax
import jax.numpy as jnp
from jax import lax
from jax.experimental import pallas as pl
from jax.experimental.pallas import tpu as pltpu
from jax.experimental.pallas import tpu_sc as plsc

CAM_LAMBDA = 3.0

B_T = 128
NUM_PATCHES = 196
EMBED_DIM = 768
SEQ = NUM_PATCHES + 1

_SC_WORKERS = 16
_ROWS_PER_W = B_T // _SC_WORKERS


def _sc_cam_gather(table, idx):
    mesh = plsc.VectorSubcoreMesh(
        core_axis_name="c", subcore_axis_name="s", num_cores=1
    )

    @functools.partial(
        pl.kernel,
        mesh=mesh,
        out_type=jax.ShapeDtypeStruct((B_T, EMBED_DIM), jnp.float32),
        scratch_types=[
            pltpu.VMEM((_ROWS_PER_W,), jnp.int32),
            pltpu.VMEM((_ROWS_PER_W, EMBED_DIM), jnp.float32),
            pltpu.SemaphoreType.DMA,
        ],
    )
    def gather_kernel(table_hbm, idx_hbm, out_hbm, idx_v, rows_v, sem):
        base = lax.axis_index("s") * _ROWS_PER_W
        pltpu.sync_copy(idx_hbm.at[pl.ds(base, _ROWS_PER_W)], idx_v)
        pltpu.async_copy(table_hbm.at[idx_v], rows_v, sem).wait()
        pltpu.sync_copy(rows_v, out_hbm.at[pl.ds(base, _ROWS_PER_W)])

    return gather_kernel(table, idx)


_BB = 16


def _tc_fuse(img_t, heat_t, cam3, cls2d, pos3):
    def body(img_ref, heat_ref, cam_ref, cls_ref, pos_ref, out_ref):
        camv = (cam_ref[...] * CAM_LAMBDA)[None]
        out_ref[0:1] = cls_ref[...][None] + pos_ref[0:1] + camv
        out_ref[1:] = img_ref[...] + heat_ref[...] + pos_ref[1:] + camv

    return pl.pallas_call(
        body,
        grid=(B_T // _BB,),
        in_specs=[
            pl.BlockSpec((NUM_PATCHES, _BB, EMBED_DIM), lambda i: (0, i, 0)),
            pl.BlockSpec((NUM_PATCHES, _BB, EMBED_DIM), lambda i: (0, i, 0)),
            pl.BlockSpec((_BB, EMBED_DIM), lambda i: (i, 0)),
            pl.BlockSpec((1, EMBED_DIM), lambda i: (0, 0)),
            pl.BlockSpec((SEQ, 1, EMBED_DIM), lambda i: (0, 0, 0)),
        ],
        out_specs=pl.BlockSpec((SEQ, _BB, EMBED_DIM), lambda i: (0, i, 0)),
        out_shape=jax.ShapeDtypeStruct((SEQ, B_T, EMBED_DIM), jnp.float32),
        compiler_params=pltpu.CompilerParams(
            dimension_semantics=("parallel",),
        ),
    )(img_t, heat_t, cam3, cls2d, pos3)


def kernel(img_tokens, heatmap_tokens, cam_labels, cls_token, pos_embed, cam):
    labels = cam_labels.astype(jnp.int32)
    cam3 = _sc_cam_gather(cam.reshape(cam.shape[0], cam.shape[-1]), labels)
    img_t = jnp.transpose(img_tokens, (1, 0, 2))
    heat_t = jnp.transpose(heatmap_tokens, (1, 0, 2))
    pos3 = pos_embed.reshape(SEQ, 1, EMBED_DIM)
    cls2d = cls_token.reshape(1, EMBED_DIM)
    out_t = _tc_fuse(img_t, heat_t, cam3, cls2d, pos3)
    return jnp.transpose(out_t, (1, 0, 2))
